# baseline (speedup 1.0000x reference)
import functools

import jax
import jax.numpy as jnp
from jax import lax
from jax.experimental import pallas as pl
from jax.experimental.pallas import tpu as pltpu
from jax.experimental.pallas import tpu_sc as plsc

SEQ = 200
EMB = 64
PADDED = 128
BATCH = 1024
VOCAB = 100000
NC = 2
NS = 16
NW = NC * NS
LANES = 16
SCALE = 8.0

B_PER_W = BATCH // NW
S_CHUNK = 8
N_CHUNK = SEQ // S_CHUNK
ROWS = S_CHUNK * B_PER_W
STR = 25


def _sc_embed(idx4, tblv, pos_table):
    mesh = plsc.VectorSubcoreMesh(
        core_axis_name="c", subcore_axis_name="s", num_cores=NC, num_subcores=NS
    )

    @functools.partial(
        pl.kernel,
        mesh=mesh,
        compiler_params=pltpu.CompilerParams(
            use_tc_tiling_on_sc=False, needs_layout_passes=False
        ),
        out_type=jax.ShapeDtypeStruct((STR * S_CHUNK, 8, 8, 8, PADDED), jnp.float32),
        scratch_types=[
            pltpu.VMEM((STR, 8, B_PER_W), jnp.int32),
            pltpu.VMEM((SEQ * B_PER_W,), jnp.int32),
            pltpu.VMEM((SEQ, EMB), jnp.float32),
            pltpu.VMEM((ROWS, EMB), jnp.float32),
            pltpu.VMEM((ROWS, EMB), jnp.float32),
            pltpu.VMEM((S_CHUNK, 8, 8, B_PER_W), jnp.float32),
            pltpu.VMEM((S_CHUNK, 8, 8, B_PER_W), jnp.float32),
            pltpu.SemaphoreType.DMA,
            pltpu.SemaphoreType.DMA,
            pltpu.SemaphoreType.DMA,
            pltpu.SemaphoreType.DMA,
            pltpu.SemaphoreType.DMA,
        ],
    )
    def k(idx_hbm, tok_hbm, pos_hbm, out_hbm, idx_st, idx2_v, pos_v,
          gb0, gb1, ob0, ob1, ssem, gs0, gs1, ws0, ws1):
        gbufs = (gb0, gb1)
        obufs = (ob0, ob1)
        gsem = (gs0, gs1)
        wsem = (ws0, ws1)
        wid = lax.axis_index("s") * NC + lax.axis_index("c")
        tb = wid // 4
        lb0 = (wid % 4) * B_PER_W

        pltpu.sync_copy(pos_hbm, pos_v)

        def idx_dma(tr, carry):
            pltpu.async_copy(
                idx_hbm.at[tr, tb, :, pl.ds(lb0, B_PER_W)], idx_st.at[tr], ssem
            )
            return carry

        lax.fori_loop(0, STR, idx_dma, 0)
        pltpu.make_async_copy(
            idx_hbm.at[:, tb, :, pl.ds(lb0, B_PER_W)], idx_st, ssem
        ).wait()

        def idx_flat(tr, carry):
            for sl in range(8):
                for h in range(B_PER_W // LANES):
                    off = tr * 256 + sl * B_PER_W + h * LANES
                    idx2_v[pl.ds(off, LANES)] = (
                        idx_st[tr, sl, pl.ds(h * LANES, LANES)] * 2
                    )
            return carry

        lax.fori_loop(0, STR, idx_flat, 0)

        def start_gather(kc, b):
            for h in range(2):
                pltpu.async_copy(
                    tok_hbm.at[idx2_v.at[pl.ds(kc * ROWS + h * 128, 128)]],
                    gbufs[b].at[pl.ds(h * 128, 128)],
                    gsem[b],
                )

        def wait_gather(b):
            pltpu.make_async_copy(tok_hbm.at[pl.ds(0, ROWS)], gbufs[b], gsem[b]).wait()

        def start_wb(kc, b):
            for td in range(8):
                pltpu.async_copy(
                    obufs[b].at[:, td],
                    out_hbm.at[pl.ds(kc * S_CHUNK, S_CHUNK), td, tb, :,
                               pl.ds(lb0, B_PER_W)],
                    wsem[b],
                )

        def wait_wb(b):
            pltpu.make_async_copy(
                obufs[b],
                out_hbm.at[pl.ds(0, S_CHUNK), :, 0, :, pl.ds(0, B_PER_W)],
                wsem[b],
            ).wait()

        iota = lax.iota(jnp.int32, LANES)

        def compute(kc, b):
            gbuf = gbufs[b]
            obuf = obufs[b]

            def r_body(r, carry):
                ss = r // 4
                bb0 = (r % 4) * 8
                prow = kc * S_CHUNK + ss
                i_ss = jnp.full((LANES,), ss, jnp.int32)
                for u in range(8):
                    row = ss * B_PER_W + bb0 + u
                    i_bb = jnp.full((LANES,), bb0 + u, jnp.int32)
                    for j in range(EMB // LANES):
                        d = j * LANES + iota
                        v = gbuf[row, pl.ds(j * LANES, LANES)] * SCALE + \
                            pos_v[prow, pl.ds(j * LANES, LANES)]
                        plsc.store_scatter(
                            obuf,
                            [i_ss, d >> 3, d & 7, i_bb],
                            v,
                        )
                return carry

            lax.fori_loop(0, ROWS // 8, r_body, 0)

        start_gather(0, 0)

        def outer(o, carry):
            for phase in range(2):
                kc = 2 * o + phase
                b = phase
                nb = 1 - phase
                start_gather(kc + 1, nb)
                wait_gather(b)

                @pl.when(o >= 1)
                def _():
                    wait_wb(b)

                compute(kc, b)
                start_wb(kc, b)
            return carry

        lax.fori_loop(0, (N_CHUNK - 1) // 2, outer, 0)
        wait_gather(0)
        wait_wb(0)
        compute(N_CHUNK - 1, 0)
        start_wb(N_CHUNK - 1, 0)
        wait_wb(1)
        wait_wb(0)

    return k(idx4, tblv, pos_table)


def kernel(inputs, token_table, pos_table):
    idx4 = (
        inputs.astype(jnp.int32)
        .T.reshape(STR, 8, 8, PADDED)
        .transpose(0, 2, 1, 3)
    )
    tbl128 = jnp.pad(token_table.astype(jnp.float32), ((0, 0), (0, PADDED - EMB)))
    tblv = tbl128.reshape(2 * VOCAB, EMB)
    out5 = _sc_embed(idx4, tblv, pos_table.astype(jnp.float32))
    return jnp.transpose(out5, (2, 4, 0, 1, 3)).reshape(BATCH, SEQ, EMB)
